# baseline (speedup 1.0000x reference)
import functools

import jax
import jax.numpy as jnp
from jax import lax
from jax.experimental import pallas as pl
from jax.experimental.pallas import tpu as pltpu
from jax.experimental.pallas import tpu_sc as plsc

B = 32
N = 576
C = 81
PADC = 128
SLAB = N * PADC
K = 100
NV_MAIN = N * 5
NV80 = N // 16
NV = NV_MAIN + NV80
CAND_CAP = 128
CAND_PAD = 144
FLAGS_PAD = 2928
OUTW = 104
NUM_CORES = 2
NUM_SUBCORES = 16

_L1_SHIFT = 21
_L2_SHIFT = 12
_L1_BUCKETS = 512
_L2_BUCKETS = 512
_DIV81_MUL = 25891
_DIV81_SHIFT = 21
_DIV5_MUL = 13108
_DIV5_SHIFT = 16


def _body(prob_hbm, pb_hbm, ts_hbm, outs_hbm, outl_hbm, outb_hbm,
          sc_v, hist, pb_v, ts_v, flags,
          cand_s, cand_i, cand_l, cand_b,
          outs_v, outl_v, outb_v, sem0, sem1, sem2):
    b = lax.axis_index("s") * NUM_CORES + lax.axis_index("c")

    cp_prob = pltpu.async_copy(prob_hbm.at[pl.ds(b * SLAB, SLAB)], sc_v,
                               sem0)
    cp_pb = pltpu.async_copy(pb_hbm.at[b], pb_v, sem1)
    cp_ts = pltpu.async_copy(ts_hbm, ts_v.at[pl.ds(0, B * 2)], sem2)

    lane = jnp.arange(16, dtype=jnp.int32)
    ones = jnp.ones((16,), jnp.int32)
    col80 = jnp.full((16,), C - 1, jnp.int32)

    @plsc.parallel_loop(0, ((_L1_BUCKETS + _L2_BUCKETS) * 16) // 16,
                        unroll=8)
    def _(i):
        hist[pl.ds(i * 16, 16)] = jnp.zeros((16,), jnp.int32)

    flags[pl.ds(FLAGS_PAD - 16, 16)] = jnp.zeros((16,), jnp.int32)

    cp_prob.wait()

    @plsc.parallel_loop(0, N, unroll=8)
    def _(r):
        for u in range(5):
            s = sc_v[pl.ds(r * PADC + u * 16, 16)]
            bits = plsc.bitcast(s, jnp.int32)
            slot = ((bits >> _L1_SHIFT) << 4) + lane
            plsc.addupdate_scatter(hist, [slot], ones)

    @plsc.parallel_loop(0, NV80, unroll=4)
    def _(i):
        rvec = i * 16 + lane
        s = plsc.load_gather(sc_v, [rvec * PADC + col80])
        bits = plsc.bitcast(s, jnp.int32)
        slot = ((bits >> _L1_SHIFT) << 4) + lane
        plsc.addupdate_scatter(hist, [slot], ones)

    def bucket_count(t):
        return jnp.sum(hist[pl.ds(t * 16, 16)])

    def scan_cond(carry):
        return carry[1] < K

    def scan_body(carry):
        t, acc, _ = carry
        return (t - 1, acc + bucket_count(t), acc)

    t1, _, cnt_hi = lax.while_loop(
        scan_cond, scan_body, (jnp.int32(_L1_BUCKETS - 1), jnp.int32(0),
                               jnp.int32(0)))
    T = t1 + 1

    @plsc.parallel_loop(0, N, unroll=8)
    def _(r):
        for u in range(5):
            s = sc_v[pl.ds(r * PADC + u * 16, 16)]
            bits = plsc.bitcast(s, jnp.int32)
            inb = (bits >> _L1_SHIFT) == T
            slot = ((((bits >> _L2_SHIFT) & (_L2_BUCKETS - 1))) << 4) + lane
            plsc.addupdate_scatter(hist, [slot + _L1_BUCKETS * 16], ones,
                                   mask=inb)

    @plsc.parallel_loop(0, NV80, unroll=4)
    def _(i):
        rvec = i * 16 + lane
        s = plsc.load_gather(sc_v, [rvec * PADC + col80])
        bits = plsc.bitcast(s, jnp.int32)
        inb = (bits >> _L1_SHIFT) == T
        slot = ((((bits >> _L2_SHIFT) & (_L2_BUCKETS - 1))) << 4) + lane
        plsc.addupdate_scatter(hist, [slot + _L1_BUCKETS * 16], ones,
                               mask=inb)

    def scan2_cond(carry):
        return cnt_hi + carry[1] < K

    t2, _, _ = lax.while_loop(
        scan2_cond, scan_body,
        (jnp.int32(_L1_BUCKETS + _L2_BUCKETS - 1), jnp.int32(0),
         jnp.int32(0)))
    T2 = t2 + 1 - _L1_BUCKETS

    thresh = (T << _L1_SHIFT) | (T2 << _L2_SHIFT)

    @plsc.parallel_loop(0, N, unroll=8)
    def _(r):
        for u in range(5):
            s = sc_v[pl.ds(r * PADC + u * 16, 16)]
            bits = plsc.bitcast(s, jnp.int32)
            cntv = plsc.all_reduce_population_count(bits >= thresh)
            plsc.store_scatter(flags, [jnp.full((16,), r * 5 + u,
                                                jnp.int32)],
                               cntv, mask=lane == 0)

    @plsc.parallel_loop(0, NV80, unroll=4)
    def _(i):
        rvec = i * 16 + lane
        s = plsc.load_gather(sc_v, [rvec * PADC + col80])
        bits = plsc.bitcast(s, jnp.int32)
        cntv = plsc.all_reduce_population_count(bits >= thresh)
        plsc.store_scatter(flags, [jnp.full((16,), NV_MAIN + i, jnp.int32)],
                           cntv, mask=lane == 0)

    def compact_main(fv, off):
        f = flags[pl.ds(fv * 16, 16)]
        inc = plsc.cumsum(f)
        excl = inc - f
        tot = inc[15]

        @pl.when(tot > 0)
        def _():
            for l in range(16):
                @pl.when(f[l] > 0)
                def _(l=l):
                    c = fv * 16 + l
                    r = (c * _DIV5_MUL) >> _DIV5_SHIFT
                    u16 = (c - r * 5) * 16
                    dst = jnp.minimum(off + excl[l], CAND_CAP)
                    s = sc_v[pl.ds(r * PADC + u16, 16)]
                    bits = plsc.bitcast(s, jnp.int32)
                    m = bits >= thresh
                    plsc.store_compressed(cand_s.at[pl.ds(dst, 16)], s,
                                          mask=m)
                    plsc.store_compressed(cand_i.at[pl.ds(dst, 16)],
                                          r * C + u16 + lane, mask=m)
        return off + tot
    off0 = lax.fori_loop(0, NV_MAIN // 16, compact_main, jnp.int32(0))

    def compact_c80(fv, off):
        f = flags[pl.ds(NV_MAIN + fv * 16, 16)]
        inc = plsc.cumsum(f)
        excl = inc - f
        tot = inc[15]

        @pl.when(tot > 0)
        def _():
            for l in range(16):
                @pl.when(f[l] > 0)
                def _(l=l):
                    i = fv * 16 + l
                    rvec = i * 16 + lane
                    dst = jnp.minimum(off + excl[l], CAND_CAP)
                    s = plsc.load_gather(sc_v, [rvec * PADC + col80])
                    bits = plsc.bitcast(s, jnp.int32)
                    m = bits >= thresh
                    plsc.store_compressed(cand_s.at[pl.ds(dst, 16)], s,
                                          mask=m)
                    plsc.store_compressed(cand_i.at[pl.ds(dst, 16)],
                                          rvec * C + (C - 1), mask=m)
        return off + tot
    n_cand = lax.fori_loop(0, (FLAGS_PAD - NV_MAIN) // 16, compact_c80, off0)
    n_cand = jnp.minimum(n_cand, CAND_CAP)

    for v in range(CAND_CAP // 16):
        g = v * 16 + lane
        m = g >= n_cand
        sv = jnp.where(m, jnp.float32(-1.0), cand_s[pl.ds(v * 16, 16)])
        iv = jnp.where(m, 0, cand_i[pl.ds(v * 16, 16)])
        cand_s[pl.ds(v * 16, 16)] = sv
        cand_i[pl.ds(v * 16, 16)] = iv

    cp_pb.wait()
    cp_ts.wait()
    n_cvec = (n_cand + 15) >> 4
    tsv = ts_v[pl.ds(2 * b, 16)]
    th = tsv[0].astype(jnp.float32)
    tw = tsv[1].astype(jnp.float32)
    img = jnp.maximum(th, tw)
    for v in range(CAND_CAP // 16):
        @pl.when(v < n_cvec)
        def _(v=v):
            iv = cand_i[pl.ds(v * 16, 16)]
            n = (iv * _DIV81_MUL) >> _DIV81_SHIFT
            cand_l[pl.ds(v * 16, 16)] = iv - n * C
            base = n * 4
            cx = plsc.load_gather(pb_v, [base])
            cy = plsc.load_gather(pb_v, [base + 1])
            w = plsc.load_gather(pb_v, [base + 2])
            h = plsc.load_gather(pb_v, [base + 3])
            x0 = jnp.clip((cx - 0.5 * w) * img, 0.0, tw)
            y0 = jnp.clip((cy - 0.5 * h) * img, 0.0, th)
            x1 = jnp.clip((cx + 0.5 * w) * img, 0.0, tw)
            y1 = jnp.clip((cy + 0.5 * h) * img, 0.0, th)
            cand_b[pl.ds(0 * CAND_PAD + v * 16, 16)] = x0
            cand_b[pl.ds(1 * CAND_PAD + v * 16, 16)] = y0
            cand_b[pl.ds(2 * CAND_PAD + v * 16, 16)] = x1
            cand_b[pl.ds(3 * CAND_PAD + v * 16, 16)] = y1

    for v in range(CAND_CAP // 16):
        @pl.when(v < n_cvec)
        def _(v=v):
            si = cand_s[pl.ds(v * 16, 16)]
            ii = cand_i[pl.ds(v * 16, 16)]

            def rank_step(jv, acc, si=si, ii=ii):
                sv = cand_s[pl.ds(jv * 16, 16)]
                ivv = cand_i[pl.ds(jv * 16, 16)]
                for l in range(16):
                    sj = sv[l]
                    ij = ivv[l]
                    beats = (sj > si) | ((sj == si) & (ij < ii))
                    acc = acc + beats.astype(jnp.int32)
                return acc
            rank = lax.fori_loop(0, n_cvec, rank_step,
                                 jnp.zeros((16,), jnp.int32))
            m = rank < K
            plsc.store_scatter(outs_v, [rank], si, mask=m)
            plsc.store_scatter(outl_v, [rank], cand_l[pl.ds(v * 16, 16)],
                               mask=m)
            for c in range(4):
                plsc.store_scatter(outb_v, [rank * 4 + c],
                                   cand_b[pl.ds(c * CAND_PAD + v * 16, 16)],
                                   mask=m)

    pltpu.sync_copy(outs_v, outs_hbm.at[b])
    pltpu.sync_copy(outl_v, outl_hbm.at[b])
    pltpu.sync_copy(outb_v, outb_hbm.at[b])


@jax.jit
def kernel(logits, obj, pred_boxes, target_sizes):
    prob = jax.nn.sigmoid(logits)
    prob = prob.at[..., -1].multiply(obj)
    prob_pad = jnp.pad(prob, ((0, 0), (0, 0), (0, PADC - C)))
    prob_1d = prob_pad.reshape(B * SLAB)
    pb_flat = pred_boxes.reshape(B, N * 4)
    ts_flat = target_sizes.astype(jnp.int32).reshape(B * 2)

    mesh = plsc.VectorSubcoreMesh(core_axis_name="c", subcore_axis_name="s",
                                  num_cores=NUM_CORES,
                                  num_subcores=NUM_SUBCORES)
    run = functools.partial(
        pl.kernel,
        out_type=[
            jax.ShapeDtypeStruct((B, OUTW), jnp.float32),
            jax.ShapeDtypeStruct((B, OUTW), jnp.int32),
            jax.ShapeDtypeStruct((B, K * 4), jnp.float32),
        ],
        mesh=mesh,
        compiler_params=pltpu.CompilerParams(needs_layout_passes=False),
        scratch_types=[
            pltpu.VMEM((SLAB,), jnp.float32),
            pltpu.VMEM(((_L1_BUCKETS + _L2_BUCKETS) * 16,), jnp.int32),
            pltpu.VMEM((N * 4,), jnp.float32),
            pltpu.VMEM((80,), jnp.int32),
            pltpu.VMEM((FLAGS_PAD,), jnp.int32),
            pltpu.VMEM((CAND_PAD,), jnp.float32),
            pltpu.VMEM((CAND_PAD,), jnp.int32),
            pltpu.VMEM((CAND_PAD,), jnp.int32),
            pltpu.VMEM((4 * CAND_PAD,), jnp.float32),
            pltpu.VMEM((OUTW,), jnp.float32),
            pltpu.VMEM((OUTW,), jnp.int32),
            pltpu.VMEM((K * 4,), jnp.float32),
            pltpu.SemaphoreType.DMA,
            pltpu.SemaphoreType.DMA,
            pltpu.SemaphoreType.DMA,
        ],
    )(_body)
    outs, outl, outb = run(prob_1d, pb_flat, ts_flat)
    return outs[:, :K], outl[:, :K], outb.reshape(B, K, 4)


# speedup vs baseline: 1.0128x; 1.0128x over previous
import functools

import jax
import jax.numpy as jnp
from jax import lax
from jax.experimental import pallas as pl
from jax.experimental.pallas import tpu as pltpu
from jax.experimental.pallas import tpu_sc as plsc

B = 32
N = 576
C = 81
PADC = 128
SLAB = N * PADC
K = 100
NV_MAIN = N * 5
NV80 = N // 16
NV = NV_MAIN + NV80
CAND_CAP = 128
CAND_PAD = 144
FLAGS_PAD = 2928
OUTW = 104
NUM_CORES = 2
NUM_SUBCORES = 16

_L1_SHIFT = 21
_L2_SHIFT = 12
_L1_BUCKETS = 512
_L2_BUCKETS = 512
_DIV81_MUL = 25891
_DIV81_SHIFT = 21
_DIV5_MUL = 13108
_DIV5_SHIFT = 16


def _body(prob_hbm, pb_hbm, ts_hbm, outs_hbm, outl_hbm, outb_hbm,
          sc_v, hist, pb_v, ts_v, flags,
          cand_s, cand_i, cand_l, cand_b,
          outs_v, outl_v, outb_v, sem0, sem1, sem2):
    b = lax.axis_index("s") * NUM_CORES + lax.axis_index("c")

    cp_prob = pltpu.async_copy(prob_hbm.at[pl.ds(b * SLAB, SLAB)], sc_v,
                               sem0)
    cp_pb = pltpu.async_copy(pb_hbm.at[b], pb_v, sem1)
    cp_ts = pltpu.async_copy(ts_hbm, ts_v.at[pl.ds(0, B * 2)], sem2)

    lane = jnp.arange(16, dtype=jnp.int32)
    ones = jnp.ones((16,), jnp.int32)
    col80 = jnp.full((16,), C - 1, jnp.int32)

    @plsc.parallel_loop(0, ((_L1_BUCKETS + _L2_BUCKETS) * 16) // 16,
                        unroll=8)
    def _(i):
        hist[pl.ds(i * 16, 16)] = jnp.zeros((16,), jnp.int32)

    flags[pl.ds(FLAGS_PAD - 16, 16)] = jnp.zeros((16,), jnp.int32)

    cp_prob.wait()

    @plsc.parallel_loop(0, N, unroll=4)
    def _(r):
        for u in range(5):
            s = sc_v[pl.ds(r * PADC + u * 16, 16)]
            bits = plsc.bitcast(s, jnp.int32)
            slot = ((bits >> _L1_SHIFT) << 4) + lane
            plsc.addupdate_scatter(hist, [slot], ones)

    @plsc.parallel_loop(0, NV80, unroll=4)
    def _(i):
        rvec = i * 16 + lane
        s = plsc.load_gather(sc_v, [rvec * PADC + col80])
        bits = plsc.bitcast(s, jnp.int32)
        slot = ((bits >> _L1_SHIFT) << 4) + lane
        plsc.addupdate_scatter(hist, [slot], ones)

    def bucket_count(t):
        return jnp.sum(hist[pl.ds(t * 16, 16)])

    def scan_cond(carry):
        return carry[1] < K

    def scan_body(carry):
        t, acc, _ = carry
        return (t - 1, acc + bucket_count(t), acc)

    t1, _, cnt_hi = lax.while_loop(
        scan_cond, scan_body, (jnp.int32(_L1_BUCKETS - 1), jnp.int32(0),
                               jnp.int32(0)))
    T = t1 + 1

    @plsc.parallel_loop(0, N, unroll=4)
    def _(r):
        for u in range(5):
            s = sc_v[pl.ds(r * PADC + u * 16, 16)]
            bits = plsc.bitcast(s, jnp.int32)
            inb = (bits >> _L1_SHIFT) == T
            slot = ((((bits >> _L2_SHIFT) & (_L2_BUCKETS - 1))) << 4) + lane
            plsc.addupdate_scatter(hist, [slot + _L1_BUCKETS * 16], ones,
                                   mask=inb)

    @plsc.parallel_loop(0, NV80, unroll=4)
    def _(i):
        rvec = i * 16 + lane
        s = plsc.load_gather(sc_v, [rvec * PADC + col80])
        bits = plsc.bitcast(s, jnp.int32)
        inb = (bits >> _L1_SHIFT) == T
        slot = ((((bits >> _L2_SHIFT) & (_L2_BUCKETS - 1))) << 4) + lane
        plsc.addupdate_scatter(hist, [slot + _L1_BUCKETS * 16], ones,
                               mask=inb)

    def scan2_cond(carry):
        return cnt_hi + carry[1] < K

    t2, _, _ = lax.while_loop(
        scan2_cond, scan_body,
        (jnp.int32(_L1_BUCKETS + _L2_BUCKETS - 1), jnp.int32(0),
         jnp.int32(0)))
    T2 = t2 + 1 - _L1_BUCKETS

    thresh = (T << _L1_SHIFT) | (T2 << _L2_SHIFT)

    @plsc.parallel_loop(0, N, unroll=4)
    def _(r):
        cnt_row = jnp.zeros((16,), jnp.int32)
        for u in range(5):
            s = sc_v[pl.ds(r * PADC + u * 16, 16)]
            bits = plsc.bitcast(s, jnp.int32)
            cntv = plsc.all_reduce_population_count(bits >= thresh)
            cnt_row = jnp.where(lane == u, cntv, cnt_row)
        plsc.store_scatter(flags, [r * 5 + lane], cnt_row, mask=lane < 5)

    @plsc.parallel_loop(0, NV80, unroll=4)
    def _(i):
        rvec = i * 16 + lane
        s = plsc.load_gather(sc_v, [rvec * PADC + col80])
        bits = plsc.bitcast(s, jnp.int32)
        cntv = plsc.all_reduce_population_count(bits >= thresh)
        plsc.store_scatter(flags, [jnp.full((16,), NV_MAIN + i, jnp.int32)],
                           cntv, mask=lane == 0)

    def compact_main(fv, off):
        f = flags[pl.ds(fv * 16, 16)]
        inc = plsc.cumsum(f)
        excl = inc - f
        tot = inc[15]

        @pl.when(tot > 0)
        def _():
            for l in range(16):
                @pl.when(f[l] > 0)
                def _(l=l):
                    c = fv * 16 + l
                    r = (c * _DIV5_MUL) >> _DIV5_SHIFT
                    u16 = (c - r * 5) * 16
                    dst = jnp.minimum(off + excl[l], CAND_CAP)
                    s = sc_v[pl.ds(r * PADC + u16, 16)]
                    bits = plsc.bitcast(s, jnp.int32)
                    m = bits >= thresh
                    plsc.store_compressed(cand_s.at[pl.ds(dst, 16)], s,
                                          mask=m)
                    plsc.store_compressed(cand_i.at[pl.ds(dst, 16)],
                                          r * C + u16 + lane, mask=m)
        return off + tot
    off0 = lax.fori_loop(0, NV_MAIN // 16, compact_main, jnp.int32(0))

    def compact_c80(fv, off):
        f = flags[pl.ds(NV_MAIN + fv * 16, 16)]
        inc = plsc.cumsum(f)
        excl = inc - f
        tot = inc[15]

        @pl.when(tot > 0)
        def _():
            for l in range(16):
                @pl.when(f[l] > 0)
                def _(l=l):
                    i = fv * 16 + l
                    rvec = i * 16 + lane
                    dst = jnp.minimum(off + excl[l], CAND_CAP)
                    s = plsc.load_gather(sc_v, [rvec * PADC + col80])
                    bits = plsc.bitcast(s, jnp.int32)
                    m = bits >= thresh
                    plsc.store_compressed(cand_s.at[pl.ds(dst, 16)], s,
                                          mask=m)
                    plsc.store_compressed(cand_i.at[pl.ds(dst, 16)],
                                          rvec * C + (C - 1), mask=m)
        return off + tot
    n_cand = lax.fori_loop(0, (FLAGS_PAD - NV_MAIN) // 16, compact_c80, off0)
    n_cand = jnp.minimum(n_cand, CAND_CAP)

    for v in range(CAND_CAP // 16):
        g = v * 16 + lane
        m = g >= n_cand
        sv = jnp.where(m, jnp.float32(-1.0), cand_s[pl.ds(v * 16, 16)])
        iv = jnp.where(m, 0, cand_i[pl.ds(v * 16, 16)])
        cand_s[pl.ds(v * 16, 16)] = sv
        cand_i[pl.ds(v * 16, 16)] = iv

    cp_pb.wait()
    cp_ts.wait()
    n_cvec = (n_cand + 15) >> 4
    tsv = ts_v[pl.ds(2 * b, 16)]
    th = tsv[0].astype(jnp.float32)
    tw = tsv[1].astype(jnp.float32)
    img = jnp.maximum(th, tw)
    for v in range(CAND_CAP // 16):
        @pl.when(v < n_cvec)
        def _(v=v):
            iv = cand_i[pl.ds(v * 16, 16)]
            n = (iv * _DIV81_MUL) >> _DIV81_SHIFT
            cand_l[pl.ds(v * 16, 16)] = iv - n * C
            base = n * 4
            cx = plsc.load_gather(pb_v, [base])
            cy = plsc.load_gather(pb_v, [base + 1])
            w = plsc.load_gather(pb_v, [base + 2])
            h = plsc.load_gather(pb_v, [base + 3])
            x0 = jnp.clip((cx - 0.5 * w) * img, 0.0, tw)
            y0 = jnp.clip((cy - 0.5 * h) * img, 0.0, th)
            x1 = jnp.clip((cx + 0.5 * w) * img, 0.0, tw)
            y1 = jnp.clip((cy + 0.5 * h) * img, 0.0, th)
            cand_b[pl.ds(0 * CAND_PAD + v * 16, 16)] = x0
            cand_b[pl.ds(1 * CAND_PAD + v * 16, 16)] = y0
            cand_b[pl.ds(2 * CAND_PAD + v * 16, 16)] = x1
            cand_b[pl.ds(3 * CAND_PAD + v * 16, 16)] = y1

    for v in range(CAND_CAP // 16):
        @pl.when(v < n_cvec)
        def _(v=v):
            si = cand_s[pl.ds(v * 16, 16)]
            ii = cand_i[pl.ds(v * 16, 16)]

            def rank_step(jv, acc, si=si, ii=ii):
                sv = cand_s[pl.ds(jv * 16, 16)]
                ivv = cand_i[pl.ds(jv * 16, 16)]
                for l in range(16):
                    sj = sv[l]
                    ij = ivv[l]
                    beats = (sj > si) | ((sj == si) & (ij < ii))
                    acc = acc + beats.astype(jnp.int32)
                return acc
            rank = lax.fori_loop(0, n_cvec, rank_step,
                                 jnp.zeros((16,), jnp.int32))
            m = rank < K
            plsc.store_scatter(outs_v, [rank], si, mask=m)
            plsc.store_scatter(outl_v, [rank], cand_l[pl.ds(v * 16, 16)],
                               mask=m)
            for c in range(4):
                plsc.store_scatter(outb_v, [rank * 4 + c],
                                   cand_b[pl.ds(c * CAND_PAD + v * 16, 16)],
                                   mask=m)

    pltpu.sync_copy(outs_v, outs_hbm.at[b])
    pltpu.sync_copy(outl_v, outl_hbm.at[b])
    pltpu.sync_copy(outb_v, outb_hbm.at[b])


@jax.jit
def kernel(logits, obj, pred_boxes, target_sizes):
    prob = jax.nn.sigmoid(logits)
    prob = prob.at[..., -1].multiply(obj)
    prob_pad = jnp.pad(prob, ((0, 0), (0, 0), (0, PADC - C)))
    prob_1d = prob_pad.reshape(B * SLAB)
    pb_flat = pred_boxes.reshape(B, N * 4)
    ts_flat = target_sizes.astype(jnp.int32).reshape(B * 2)

    mesh = plsc.VectorSubcoreMesh(core_axis_name="c", subcore_axis_name="s",
                                  num_cores=NUM_CORES,
                                  num_subcores=NUM_SUBCORES)
    run = functools.partial(
        pl.kernel,
        out_type=[
            jax.ShapeDtypeStruct((B, OUTW), jnp.float32),
            jax.ShapeDtypeStruct((B, OUTW), jnp.int32),
            jax.ShapeDtypeStruct((B, K * 4), jnp.float32),
        ],
        mesh=mesh,
        compiler_params=pltpu.CompilerParams(needs_layout_passes=False),
        scratch_types=[
            pltpu.VMEM((SLAB,), jnp.float32),
            pltpu.VMEM(((_L1_BUCKETS + _L2_BUCKETS) * 16,), jnp.int32),
            pltpu.VMEM((N * 4,), jnp.float32),
            pltpu.VMEM((80,), jnp.int32),
            pltpu.VMEM((FLAGS_PAD,), jnp.int32),
            pltpu.VMEM((CAND_PAD,), jnp.float32),
            pltpu.VMEM((CAND_PAD,), jnp.int32),
            pltpu.VMEM((CAND_PAD,), jnp.int32),
            pltpu.VMEM((4 * CAND_PAD,), jnp.float32),
            pltpu.VMEM((OUTW,), jnp.float32),
            pltpu.VMEM((OUTW,), jnp.int32),
            pltpu.VMEM((K * 4,), jnp.float32),
            pltpu.SemaphoreType.DMA,
            pltpu.SemaphoreType.DMA,
            pltpu.SemaphoreType.DMA,
        ],
    )(_body)
    outs, outl, outb = run(prob_1d, pb_flat, ts_flat)
    return outs[:, :K], outl[:, :K], outb.reshape(B, K, 4)
